# baseline (speedup 1.0000x reference)
import functools

import jax
import jax.numpy as jnp
from jax import lax
from jax.experimental import pallas as pl
from jax.experimental.pallas import tpu as pltpu
from jax.experimental.pallas import tpu_sc as plsc

_B = 16384
_D = 64
_NEG = 8
_NC = 2
_NS = 16
_NW = _NC * _NS
_CH = 512
_BI = _B // _NW
_BJ = _B * _NEG // _NW
_JCH = _BJ // _CH


def _gather_triplets(idx_i, idx_k, idx_j, table):
    mesh = plsc.VectorSubcoreMesh(core_axis_name="c", subcore_axis_name="s")

    @functools.partial(
        pl.kernel,
        mesh=mesh,
        out_type=(
            jax.ShapeDtypeStruct((_B, _D), jnp.float32),
            jax.ShapeDtypeStruct((_B, _D), jnp.float32),
            jax.ShapeDtypeStruct((_B * _NEG, _D), jnp.float32),
        ),
        scratch_types=[
            pltpu.VMEM((_CH,), jnp.int32),
            pltpu.VMEM((_CH, _D), jnp.float32),
            pltpu.SemaphoreType.DMA,
        ],
        compiler_params=pltpu.CompilerParams(use_tc_tiling_on_sc=False),
    )
    def body(idx_i_hbm, idx_k_hbm, idx_j_hbm, table_hbm,
             vi_hbm, vk_hbm, vj_hbm, idx_v, rows_v, sem):
        wid = lax.axis_index("s") * _NC + lax.axis_index("c")
        base = wid * _BI

        def one_chunk(idx_hbm, out_hbm, off):
            pltpu.sync_copy(idx_hbm.at[pl.ds(off, _CH)], idx_v)
            pltpu.async_copy(table_hbm.at[idx_v], rows_v, sem).wait()
            pltpu.sync_copy(rows_v, out_hbm.at[pl.ds(off, _CH)])

        one_chunk(idx_i_hbm, vi_hbm, base)
        one_chunk(idx_k_hbm, vk_hbm, base)
        jbase = wid * _BJ
        for c in range(_JCH):
            one_chunk(idx_j_hbm, vj_hbm, jbase + c * _CH)

    return body(idx_i, idx_k, idx_j, table)


def kernel(items, table):
    items = items.astype(jnp.int32)
    idx_i = items[:, 0]
    idx_k = items[:, 1]
    idx_j = items[:, 2:].reshape(-1)
    vi, vk, vj = _gather_triplets(idx_i, idx_k, idx_j, table)
    return vi, vk, vj.reshape(_B, _NEG, _D)


# speedup vs baseline: 1.0072x; 1.0072x over previous
import functools

import jax
import jax.numpy as jnp
from jax import lax
from jax.experimental import pallas as pl
from jax.experimental.pallas import tpu as pltpu
from jax.experimental.pallas import tpu_sc as plsc

_B = 16384
_D = 64
_NEG = 8
_NC = 2
_NS = 16
_NW = _NC * _NS
_CH = 512
_BI = _B // _NW
_BJ = _B * _NEG // _NW
_NCHUNK = 2 + _BJ // _CH
_NBUF = 3


def _gather_triplets(idx_i, idx_k, idx_j, table):
    mesh = plsc.VectorSubcoreMesh(core_axis_name="c", subcore_axis_name="s")

    @functools.partial(
        pl.kernel,
        mesh=mesh,
        out_type=(
            jax.ShapeDtypeStruct((_B, _D), jnp.float32),
            jax.ShapeDtypeStruct((_B, _D), jnp.float32),
            jax.ShapeDtypeStruct((_B * _NEG, _D), jnp.float32),
        ),
        scratch_types=(
            [pltpu.VMEM((_CH,), jnp.int32) for _ in range(_NCHUNK)]
            + [pltpu.VMEM((_CH, _D), jnp.float32) for _ in range(_NBUF)]
            + [pltpu.SemaphoreType.DMA for _ in range(2 * _NBUF + 1)]
        ),
        compiler_params=pltpu.CompilerParams(use_tc_tiling_on_sc=False),
    )
    def body(idx_i_hbm, idx_k_hbm, idx_j_hbm, table_hbm,
             vi_hbm, vk_hbm, vj_hbm, *scratch):
        idx_v = scratch[:_NCHUNK]
        bufs = scratch[_NCHUNK:_NCHUNK + _NBUF]
        gsem = scratch[_NCHUNK + _NBUF:_NCHUNK + 2 * _NBUF]
        wsem = scratch[_NCHUNK + 2 * _NBUF:_NCHUNK + 3 * _NBUF]
        isem = scratch[_NCHUNK + 3 * _NBUF]

        wid = lax.axis_index("s") * _NC + lax.axis_index("c")
        base = wid * _BI
        jbase = wid * _BJ

        chunks = [
            (idx_i_hbm, base, vi_hbm, base),
            (idx_k_hbm, base, vk_hbm, base),
        ] + [
            (idx_j_hbm, jbase + c * _CH, vj_hbm, jbase + c * _CH)
            for c in range(_BJ // _CH)
        ]

        ih = [
            pltpu.async_copy(src.at[pl.ds(ioff, _CH)], idx_v[t], isem)
            for t, (src, ioff, _, _) in enumerate(chunks)
        ]
        for h in ih:
            h.wait()

        gh, wh = {}, {}
        for t in range(_NCHUNK + 1):
            if t < _NCHUNK:
                b = t % _NBUF
                if t >= _NBUF:
                    wh[t - _NBUF].wait()
                gh[t] = pltpu.async_copy(
                    table_hbm.at[idx_v[t]], bufs[b], gsem[b])
            u = t - 1
            if 0 <= u < _NCHUNK:
                b = u % _NBUF
                _, _, dst, ooff = chunks[u]
                gh[u].wait()
                wh[u] = pltpu.async_copy(
                    bufs[b], dst.at[pl.ds(ooff, _CH)], wsem[b])
        for u in range(_NCHUNK - _NBUF, _NCHUNK):
            wh[u].wait()

    return body(idx_i, idx_k, idx_j, table)


def kernel(items, table):
    items = items.astype(jnp.int32)
    idx_i = items[:, 0]
    idx_k = items[:, 1]
    idx_j = items[:, 2:].reshape(-1)
    vi, vk, vj = _gather_triplets(idx_i, idx_k, idx_j, table)
    return vi, vk, vj.reshape(_B, _NEG, _D)
